# baseline (speedup 1.0000x reference)
import functools

import jax
import jax.numpy as jnp
from jax import lax
from jax.experimental import pallas as pl
from jax.experimental.pallas import tpu as pltpu
from jax.experimental.pallas import tpu_sc as plsc

_N = 8 * 32 * 32
_D = 32
_K = 8192
_BN = 256
_W = 2048


_C = 128


def _dist_argmin_body(x_ref, cb_ref, ids_ref):
    x = x_ref[...]
    cb = cb_ref[...]
    v = x * x
    g = ((v[:, 0:8] + v[:, 8:16]) + v[:, 16:24]) + v[:, 24:32]
    g = g[:, 0:4] + g[:, 4:8]
    g = g[:, 0:2] + g[:, 2:4]
    sq_in = g[:, 0:1] + g[:, 1:2]
    sq_cb = jnp.sum(cb * cb, axis=0, keepdims=True)
    dot2 = lax.dot_general((x + x).astype(jnp.bfloat16), cb.astype(jnp.bfloat16),
                           (((1,), (0,)), ((), ())),
                           preferred_element_type=jnp.float32)

    lane = lax.broadcasted_iota(jnp.int32, (_BN, _C), 1)
    acc_v = jnp.full((_BN, 1), jnp.inf, dtype=jnp.float32)
    acc_i = jnp.zeros((_BN, 1), dtype=jnp.int32)
    for w in range(_K // _W):
        wv = None
        wc = None
        for c in range(_W // _C):
            lo = w * _W + c * _C
            d_c = (sq_in - dot2[:, lo:lo + _C]) + sq_cb[:, lo:lo + _C]
            if wv is None:
                wv = d_c
                wc = jnp.zeros((_BN, _C), dtype=jnp.int32)
            else:
                lt = d_c < wv
                wv = jnp.where(lt, d_c, wv)
                wc = jnp.where(lt, c, wc)
        wi = w * _W + wc * _C + lane
        width = _C
        while width > 1:
            h = width // 2
            av, bv = wv[:, :h], wv[:, h:width]
            ai, bi = wi[:, :h], wi[:, h:width]
            take_b = (bv < av) | ((bv == av) & (bi < ai))
            wv = jnp.where(take_b, bv, av)
            wi = jnp.where(take_b, bi, ai)
            width = h
        take = (wv < acc_v) | ((wv == acc_v) & (wi < acc_i))
        acc_v = jnp.where(take, wv, acc_v).astype(jnp.bfloat16).astype(jnp.float32)
        acc_i = jnp.where(take, wi, acc_i)
    ids_ref[0, 0, :] = acc_i[:, 0]


def _argmin_ids(x, codebook):
    nb = _N // _BN
    ids3 = pl.pallas_call(
        _dist_argmin_body,
        grid=(nb,),
        in_specs=[
            pl.BlockSpec((_BN, _D), lambda i: (i, 0)),
            pl.BlockSpec((_D, _K), lambda i: (0, 0)),
        ],
        out_specs=pl.BlockSpec((1, 1, _BN), lambda i: (i, 0, 0)),
        out_shape=jax.ShapeDtypeStruct((nb, 1, _BN), jnp.int32),
    )(x, codebook)
    return ids3.reshape(_N)


def _sc_gather(table, ids):
    info = plsc.get_sparse_core_info()
    nc, ns = info.num_cores, info.num_subcores
    nw = nc * ns
    bpw = _N // nw
    mesh = plsc.VectorSubcoreMesh(core_axis_name="c", subcore_axis_name="s")

    @functools.partial(
        pl.kernel,
        mesh=mesh,
        compiler_params=pltpu.CompilerParams(use_tc_tiling_on_sc=False),
        out_type=jax.ShapeDtypeStruct((_N, _D), jnp.float32),
        scratch_types=[
            pltpu.VMEM((bpw,), jnp.int32),
            pltpu.VMEM((bpw, _D), jnp.float32),
            pltpu.SemaphoreType.DMA,
        ],
    )
    def gather_kernel(table_hbm, idx_hbm, out_hbm, idx_v, rows_v, sem):
        wid = lax.axis_index("s") * nc + lax.axis_index("c")
        base = wid * bpw
        pltpu.sync_copy(idx_hbm.at[pl.ds(base, bpw)], idx_v)
        pltpu.async_copy(table_hbm.at[idx_v], rows_v, sem).wait()
        pltpu.sync_copy(rows_v, out_hbm.at[pl.ds(base, bpw)])

    return gather_kernel(table, ids)


def kernel(encodings, codebook):
    b, d, h, w = encodings.shape
    x = jnp.transpose(encodings, (0, 2, 3, 1)).reshape(_N, _D)
    ids = _argmin_ids(x, codebook)
    emb = _sc_gather(codebook.T, ids)
    return jnp.transpose(emb.reshape(b, h, w, d), (0, 3, 1, 2))


# speedup vs baseline: 1.3167x; 1.3167x over previous
import functools

import jax
import jax.numpy as jnp
from jax import lax
from jax.experimental import pallas as pl
from jax.experimental.pallas import tpu as pltpu
from jax.experimental.pallas import tpu_sc as plsc

_N = 8 * 32 * 32
_D = 32
_K = 8192
_BN = 256
_W = 2048


_C = 128


def _dist_argmin_body(x_ref, cb_ref, ids_ref):
    x = x_ref[...]
    cb = cb_ref[...]
    v = x * x
    g = ((v[:, 0:8] + v[:, 8:16]) + v[:, 16:24]) + v[:, 24:32]
    g = g[:, 0:4] + g[:, 4:8]
    g = g[:, 0:2] + g[:, 2:4]
    sq_in = g[:, 0:1] + g[:, 1:2]
    sq_cb = jnp.sum(cb * cb, axis=0, keepdims=True)
    dot2 = lax.dot_general((x + x).astype(jnp.bfloat16), cb.astype(jnp.bfloat16),
                           (((1,), (0,)), ((), ())),
                           preferred_element_type=jnp.float32)

    dist = (sq_in - dot2) + sq_cb

    acc_v = jnp.full((_BN, 1), jnp.inf, dtype=jnp.float32)
    acc_i = jnp.zeros((_BN, 1), dtype=jnp.int32)
    for w in range(_K // _W):
        wv = dist[:, w * _W:(w + 1) * _W]
        m = jnp.min(wv, axis=1, keepdims=True)
        kidx = lax.broadcasted_iota(jnp.int32, wv.shape, 1) + w * _W
        mi = jnp.min(jnp.where(wv == m, kidx, _K), axis=1, keepdims=True)
        take = (m < acc_v) | ((m == acc_v) & (mi < acc_i))
        acc_v = jnp.where(take, m, acc_v).astype(jnp.bfloat16).astype(jnp.float32)
        acc_i = jnp.where(take, mi, acc_i)
    ids_ref[0, 0, :] = acc_i[:, 0]


def _argmin_ids(x, codebook):
    nb = _N // _BN
    ids3 = pl.pallas_call(
        _dist_argmin_body,
        grid=(nb,),
        in_specs=[
            pl.BlockSpec((_BN, _D), lambda i: (i, 0)),
            pl.BlockSpec((_D, _K), lambda i: (0, 0)),
        ],
        out_specs=pl.BlockSpec((1, 1, _BN), lambda i: (i, 0, 0)),
        out_shape=jax.ShapeDtypeStruct((nb, 1, _BN), jnp.int32),
    )(x, codebook)
    return ids3.reshape(_N)


def _sc_gather(table, ids):
    info = plsc.get_sparse_core_info()
    nc, ns = info.num_cores, info.num_subcores
    nw = nc * ns
    bpw = _N // nw
    mesh = plsc.VectorSubcoreMesh(core_axis_name="c", subcore_axis_name="s")

    @functools.partial(
        pl.kernel,
        mesh=mesh,
        compiler_params=pltpu.CompilerParams(use_tc_tiling_on_sc=False),
        out_type=jax.ShapeDtypeStruct((_N, _D), jnp.float32),
        scratch_types=[
            pltpu.VMEM((bpw,), jnp.int32),
            pltpu.VMEM((bpw, _D), jnp.float32),
            pltpu.SemaphoreType.DMA,
        ],
    )
    def gather_kernel(table_hbm, idx_hbm, out_hbm, idx_v, rows_v, sem):
        wid = lax.axis_index("s") * nc + lax.axis_index("c")
        base = wid * bpw
        pltpu.sync_copy(idx_hbm.at[pl.ds(base, bpw)], idx_v)
        pltpu.async_copy(table_hbm.at[idx_v], rows_v, sem).wait()
        pltpu.sync_copy(rows_v, out_hbm.at[pl.ds(base, bpw)])

    return gather_kernel(table, ids)


def kernel(encodings, codebook):
    b, d, h, w = encodings.shape
    x = jnp.transpose(encodings, (0, 2, 3, 1)).reshape(_N, _D)
    ids = _argmin_ids(x, codebook)
    emb = _sc_gather(codebook.T, ids)
    return jnp.transpose(emb.reshape(b, h, w, d), (0, 3, 1, 2))


# speedup vs baseline: 1.4598x; 1.1087x over previous
import functools

import jax
import jax.numpy as jnp
from jax import lax
from jax.experimental import pallas as pl
from jax.experimental.pallas import tpu as pltpu
from jax.experimental.pallas import tpu_sc as plsc

_N = 8 * 32 * 32
_D = 32
_K = 8192
_BN = 256
_W = 2048


_C = 128


def _dist_argmin_body(x_ref, cb_ref, ids_ref):
    x = x_ref[...]
    cb = cb_ref[...]
    v = x * x
    g = ((v[:, 0:8] + v[:, 8:16]) + v[:, 16:24]) + v[:, 24:32]
    g = g[:, 0:4] + g[:, 4:8]
    g = g[:, 0:2] + g[:, 2:4]
    sq_in = g[:, 0:1] + g[:, 1:2]
    sq_cb = jnp.sum(cb * cb, axis=0, keepdims=True)
    dot2 = lax.dot_general((x + x).astype(jnp.bfloat16), cb.astype(jnp.bfloat16),
                           (((1,), (0,)), ((), ())),
                           preferred_element_type=jnp.float32)

    dist = (sq_in - dot2) + sq_cb

    acc_v = jnp.full((_BN, 1), jnp.inf, dtype=jnp.float32)
    acc_i = jnp.zeros((_BN, 1), dtype=jnp.float32)
    kidx = lax.broadcasted_iota(jnp.int32, (_BN, _W), 1).astype(jnp.float32)
    for w in range(_K // _W):
        wv = dist[:, w * _W:(w + 1) * _W]
        m = jnp.min(wv, axis=1, keepdims=True)
        mi = jnp.min(jnp.where(wv == m, kidx, float(_W)), axis=1, keepdims=True)
        mi = mi + float(w * _W)
        take = (m < acc_v) | ((m == acc_v) & (mi < acc_i))
        acc_v = jnp.where(take, m, acc_v).astype(jnp.bfloat16).astype(jnp.float32)
        acc_i = jnp.where(take, mi, acc_i)
    ids_ref[0, 0, :] = acc_i[:, 0].astype(jnp.int32)


def _argmin_ids(x, codebook):
    nb = _N // _BN
    ids3 = pl.pallas_call(
        _dist_argmin_body,
        grid=(nb,),
        in_specs=[
            pl.BlockSpec((_BN, _D), lambda i: (i, 0)),
            pl.BlockSpec((_D, _K), lambda i: (0, 0)),
        ],
        out_specs=pl.BlockSpec((1, 1, _BN), lambda i: (i, 0, 0)),
        out_shape=jax.ShapeDtypeStruct((nb, 1, _BN), jnp.int32),
    )(x, codebook)
    return ids3.reshape(_N)


def _sc_gather(table, ids):
    info = plsc.get_sparse_core_info()
    nc, ns = info.num_cores, info.num_subcores
    nw = nc * ns
    bpw = _N // nw
    mesh = plsc.VectorSubcoreMesh(core_axis_name="c", subcore_axis_name="s")

    @functools.partial(
        pl.kernel,
        mesh=mesh,
        compiler_params=pltpu.CompilerParams(use_tc_tiling_on_sc=False),
        out_type=jax.ShapeDtypeStruct((_N, _D), jnp.float32),
        scratch_types=[
            pltpu.VMEM((bpw,), jnp.int32),
            pltpu.VMEM((bpw, _D), jnp.float32),
            pltpu.SemaphoreType.DMA,
        ],
    )
    def gather_kernel(table_hbm, idx_hbm, out_hbm, idx_v, rows_v, sem):
        wid = lax.axis_index("s") * nc + lax.axis_index("c")
        base = wid * bpw
        pltpu.sync_copy(idx_hbm.at[pl.ds(base, bpw)], idx_v)
        pltpu.async_copy(table_hbm.at[idx_v], rows_v, sem).wait()
        pltpu.sync_copy(rows_v, out_hbm.at[pl.ds(base, bpw)])

    return gather_kernel(table, ids)


def kernel(encodings, codebook):
    b, d, h, w = encodings.shape
    x = jnp.transpose(encodings, (0, 2, 3, 1)).reshape(_N, _D)
    ids = _argmin_ids(x, codebook)
    emb = _sc_gather(codebook.T, ids)
    return jnp.transpose(emb.reshape(b, h, w, d), (0, 3, 1, 2))


# speedup vs baseline: 1.6107x; 1.1034x over previous
import functools

import jax
import jax.numpy as jnp
from jax import lax
from jax.experimental import pallas as pl
from jax.experimental.pallas import tpu as pltpu
from jax.experimental.pallas import tpu_sc as plsc

_N = 8 * 32 * 32
_D = 32
_K = 8192
_BN = 512
_W = 2048


_C = 128


def _dist_argmin_body(x_ref, cb_ref, ids_ref):
    x = x_ref[...]
    cb = cb_ref[...]
    v = x * x
    g = ((v[:, 0:8] + v[:, 8:16]) + v[:, 16:24]) + v[:, 24:32]
    g = g[:, 0:4] + g[:, 4:8]
    g = g[:, 0:2] + g[:, 2:4]
    sq_in = g[:, 0:1] + g[:, 1:2]
    sq_cb = jnp.sum(cb * cb, axis=0, keepdims=True)
    dot2 = lax.dot_general((x + x).astype(jnp.bfloat16), cb.astype(jnp.bfloat16),
                           (((1,), (0,)), ((), ())),
                           preferred_element_type=jnp.float32)

    dist = (sq_in - dot2) + sq_cb

    acc_v = jnp.full((_BN, 1), jnp.inf, dtype=jnp.float32)
    acc_i = jnp.zeros((_BN, 1), dtype=jnp.float32)
    kidx = lax.broadcasted_iota(jnp.int32, (_BN, _W), 1).astype(jnp.float32)
    for w in range(_K // _W):
        wv = dist[:, w * _W:(w + 1) * _W]
        m = jnp.min(wv, axis=1, keepdims=True)
        mi = jnp.min(jnp.where(wv == m, kidx, float(_W)), axis=1, keepdims=True)
        mi = mi + float(w * _W)
        take = (m < acc_v) | ((m == acc_v) & (mi < acc_i))
        acc_v = jnp.where(take, m, acc_v).astype(jnp.bfloat16).astype(jnp.float32)
        acc_i = jnp.where(take, mi, acc_i)
    ids_ref[0, 0, :] = acc_i[:, 0].astype(jnp.int32)


def _argmin_ids(x, codebook):
    nb = _N // _BN
    ids3 = pl.pallas_call(
        _dist_argmin_body,
        grid=(nb,),
        in_specs=[
            pl.BlockSpec((_BN, _D), lambda i: (i, 0)),
            pl.BlockSpec((_D, _K), lambda i: (0, 0)),
        ],
        out_specs=pl.BlockSpec((1, 1, _BN), lambda i: (i, 0, 0)),
        out_shape=jax.ShapeDtypeStruct((nb, 1, _BN), jnp.int32),
    )(x, codebook)
    return ids3.reshape(_N)


def _sc_gather(table, ids):
    info = plsc.get_sparse_core_info()
    nc, ns = info.num_cores, info.num_subcores
    nw = nc * ns
    bpw = _N // nw
    mesh = plsc.VectorSubcoreMesh(core_axis_name="c", subcore_axis_name="s")

    @functools.partial(
        pl.kernel,
        mesh=mesh,
        compiler_params=pltpu.CompilerParams(use_tc_tiling_on_sc=False),
        out_type=jax.ShapeDtypeStruct((_N, _D), jnp.float32),
        scratch_types=[
            pltpu.VMEM((bpw,), jnp.int32),
            pltpu.VMEM((bpw, _D), jnp.float32),
            pltpu.SemaphoreType.DMA,
        ],
    )
    def gather_kernel(table_hbm, idx_hbm, out_hbm, idx_v, rows_v, sem):
        wid = lax.axis_index("s") * nc + lax.axis_index("c")
        base = wid * bpw
        pltpu.sync_copy(idx_hbm.at[pl.ds(base, bpw)], idx_v)
        pltpu.async_copy(table_hbm.at[idx_v], rows_v, sem).wait()
        pltpu.sync_copy(rows_v, out_hbm.at[pl.ds(base, bpw)])

    return gather_kernel(table, ids)


def kernel(encodings, codebook):
    b, d, h, w = encodings.shape
    x = jnp.transpose(encodings, (0, 2, 3, 1)).reshape(_N, _D)
    ids = _argmin_ids(x, codebook)
    emb = _sc_gather(codebook.T, ids)
    return jnp.transpose(emb.reshape(b, h, w, d), (0, 3, 1, 2))


# speedup vs baseline: 1.6134x; 1.0017x over previous
import functools

import jax
import jax.numpy as jnp
from jax import lax
from jax.experimental import pallas as pl
from jax.experimental.pallas import tpu as pltpu
from jax.experimental.pallas import tpu_sc as plsc

_N = 8 * 32 * 32
_D = 32
_K = 8192
_BN = 1024
_W = 2048


_C = 128


def _dist_argmin_body(x_ref, cb_ref, ids_ref):
    x = x_ref[...]
    cb = cb_ref[...]
    v = x * x
    g = ((v[:, 0:8] + v[:, 8:16]) + v[:, 16:24]) + v[:, 24:32]
    g = g[:, 0:4] + g[:, 4:8]
    g = g[:, 0:2] + g[:, 2:4]
    sq_in = g[:, 0:1] + g[:, 1:2]
    sq_cb = jnp.sum(cb * cb, axis=0, keepdims=True)
    dot2 = lax.dot_general((x + x).astype(jnp.bfloat16), cb.astype(jnp.bfloat16),
                           (((1,), (0,)), ((), ())),
                           preferred_element_type=jnp.float32)

    dist = (sq_in - dot2) + sq_cb

    acc_v = jnp.full((_BN, 1), jnp.inf, dtype=jnp.float32)
    acc_i = jnp.zeros((_BN, 1), dtype=jnp.float32)
    kidx = lax.broadcasted_iota(jnp.int32, (_BN, _W), 1).astype(jnp.float32)
    for w in range(_K // _W):
        wv = dist[:, w * _W:(w + 1) * _W]
        m = jnp.min(wv, axis=1, keepdims=True)
        mi = jnp.min(jnp.where(wv == m, kidx, float(_W)), axis=1, keepdims=True)
        mi = mi + float(w * _W)
        take = (m < acc_v) | ((m == acc_v) & (mi < acc_i))
        acc_v = jnp.where(take, m, acc_v).astype(jnp.bfloat16).astype(jnp.float32)
        acc_i = jnp.where(take, mi, acc_i)
    ids_ref[0, 0, :] = acc_i[:, 0].astype(jnp.int32)


def _argmin_ids(x, codebook):
    nb = _N // _BN
    ids3 = pl.pallas_call(
        _dist_argmin_body,
        grid=(nb,),
        in_specs=[
            pl.BlockSpec((_BN, _D), lambda i: (i, 0)),
            pl.BlockSpec((_D, _K), lambda i: (0, 0)),
        ],
        out_specs=pl.BlockSpec((1, 1, _BN), lambda i: (i, 0, 0)),
        out_shape=jax.ShapeDtypeStruct((nb, 1, _BN), jnp.int32),
    )(x, codebook)
    return ids3.reshape(_N)


def _sc_gather(table, ids):
    info = plsc.get_sparse_core_info()
    nc, ns = info.num_cores, info.num_subcores
    nw = nc * ns
    bpw = _N // nw
    mesh = plsc.VectorSubcoreMesh(core_axis_name="c", subcore_axis_name="s")

    @functools.partial(
        pl.kernel,
        mesh=mesh,
        compiler_params=pltpu.CompilerParams(use_tc_tiling_on_sc=False),
        out_type=jax.ShapeDtypeStruct((_N, _D), jnp.float32),
        scratch_types=[
            pltpu.VMEM((bpw,), jnp.int32),
            pltpu.VMEM((bpw, _D), jnp.float32),
            pltpu.SemaphoreType.DMA,
        ],
    )
    def gather_kernel(table_hbm, idx_hbm, out_hbm, idx_v, rows_v, sem):
        wid = lax.axis_index("s") * nc + lax.axis_index("c")
        base = wid * bpw
        pltpu.sync_copy(idx_hbm.at[pl.ds(base, bpw)], idx_v)
        pltpu.async_copy(table_hbm.at[idx_v], rows_v, sem).wait()
        pltpu.sync_copy(rows_v, out_hbm.at[pl.ds(base, bpw)])

    return gather_kernel(table, ids)


def kernel(encodings, codebook):
    b, d, h, w = encodings.shape
    x = jnp.transpose(encodings, (0, 2, 3, 1)).reshape(_N, _D)
    ids = _argmin_ids(x, codebook)
    emb = _sc_gather(codebook.T, ids)
    return jnp.transpose(emb.reshape(b, h, w, d), (0, 3, 1, 2))
